# baseline (speedup 1.0000x reference)
import functools

import jax
import jax.numpy as jnp
from jax import lax
from jax.experimental import pallas as pl
from jax.experimental.pallas import tpu as pltpu
from jax.experimental.pallas import tpu_sc as plsc

NC = 2
NS = 16
D = 128
P = 10240
ROWS_T = P // NS
K = 80
NROW = 3
NIDX = 4
E1T = 10000
E2T = 5120
EJP = NC * NS * E2T
NBLK1 = E1T // K
NBLK2 = E2T // K
NREP = 32


def _sc_segment_sums():
    mesh = plsc.VectorSubcoreMesh(
        core_axis_name="c", subcore_axis_name="s", num_cores=NC,
        num_subcores=NS)

    @functools.partial(
        pl.kernel,
        mesh=mesh,
        out_type=(
            jax.ShapeDtypeStruct((NC * P, D), jnp.float32),
            jax.ShapeDtypeStruct((NC * P, D), jnp.float32),
        ),
        scratch_types=[
            pltpu.VMEM_SHARED((P, D), jnp.float32),
            pltpu.VMEM_SHARED((P,), jnp.float32),
            pltpu.VMEM((NIDX, 3, K), jnp.int32),
            pltpu.VMEM((NIDX, K), jnp.float32),
            pltpu.VMEM((NROW, K, D), jnp.float32),
            pltpu.VMEM((K, D), jnp.float32),
            pltpu.VMEM((K,), jnp.float32),
            pltpu.VMEM((K,), jnp.float32),
            [pltpu.SemaphoreType.DMA] * NROW,
            [pltpu.SemaphoreType.DMA] * NROW,
            [pltpu.SemaphoreType.DMA] * NIDX,
            pltpu.SemaphoreType.DMA,
        ],
    )
    def sc_pass(pack1, pack2, wpack, emb_h, negrel_h,
                ones_h, z2d, z1d, sio, jpart, acc, deg,
                idx_ring, w_ring, rows_ring, rel_buf, ones_v,
                degc_v, gsems, ssems, psems, relsem):
        cid = lax.axis_index("c")
        sid = lax.axis_index("s")
        wid = cid * NS + sid
        rows0 = sid * ROWS_T

        pltpu.sync_copy(ones_h, ones_v)
        pltpu.sync_copy(z2d.at[pl.ds(rows0, ROWS_T)],
                        acc.at[pl.ds(rows0, ROWS_T)])
        pltpu.sync_copy(z1d.at[pl.ds(rows0, ROWS_T)],
                        deg.at[pl.ds(rows0, ROWS_T)])
        plsc.subcore_barrier()

        def drain_rows(sems, r):
            pltpu.make_async_copy(z2d.at[pl.ds(0, K)], rows_ring.at[r],
                                  sems[r]).wait()

        def drain_ones(r):
            pltpu.make_async_copy(z1d.at[pl.ds(0, K)], degc_v,
                                  ssems[r]).wait()

        def drain_rel():
            pltpu.make_async_copy(z2d.at[pl.ds(0, K)], rel_buf, relsem).wait()

        pbase1 = wid * NBLK1

        def pack1_issue(q, b):
            pltpu.async_copy(pack1.at[pbase1 + b], idx_ring.at[q], psems[q])

        def pack1_drain(q):
            pltpu.make_async_copy(pack1.at[pbase1], idx_ring.at[q],
                                  psems[q]).wait()

        def gather1(q, r):
            pltpu.async_copy(emb_h.at[idx_ring.at[q, 0]], rows_ring.at[r],
                             gsems[r])

        def rel_gather(q):
            pltpu.async_copy(negrel_h.at[idx_ring.at[q, 1]], rel_buf, relsem)

        def addrel(r):
            def body(k, c2):
                for j in range(D // 16):
                    sl = pl.ds(j * 16, 16)
                    rows_ring[r, k, sl] = rows_ring[r, k, sl] + rel_buf[k, sl]
                return c2

            lax.fori_loop(0, K, body, 0)

        def scatter1(q, r):
            pltpu.async_copy(rows_ring.at[r], acc.at[idx_ring.at[q, 2]],
                             ssems[r], add=True)
            pltpu.async_copy(ones_v, deg.at[idx_ring.at[q, 2]],
                             ssems[r], add=True)

        def it1(b, u, first, last_issue, last_pack):
            r0, r1 = u % NROW, (u + 1) % NROW
            q0, q1, q2 = u % NIDX, (u + 1) % NIDX, (u + 2) % NIDX
            if not first:
                drain_rows(ssems, r1)
                drain_ones(r1)
            if not last_issue:
                pack1_drain(q1)
                gather1(q1, r1)
            drain_rel()
            drain_rows(gsems, r0)
            addrel(r0)
            scatter1(q0, r0)
            if not last_pack:
                pack1_issue(q2, b + 2)
            if not last_issue:
                rel_gather(q1)

        pack1_issue(0, 0)
        pack1_issue(1, 1)
        pack1_drain(0)
        gather1(0, 0)
        rel_gather(0)
        it1(0, 0, True, False, False)
        it1(1, 1, True, False, False)

        M1 = (NBLK1 - 4) // 12 * 12

        def body1(b2, carry):
            base = 2 + b2 * 12
            for u in range(12):
                it1(base + u, 2 + u, False, False, False)
            return carry

        lax.fori_loop(0, M1 // 12, body1, 0)
        for b in range(2 + M1, NBLK1):
            it1(b, b, False, b + 1 >= NBLK1, b + 2 >= NBLK1)
        for b in (NBLK1 - 2, NBLK1 - 1):
            drain_rows(ssems, b % NROW)
            drain_ones(b % NROW)
        plsc.subcore_barrier()

        def flushc(c, carry):
            r = rows0 + c * K
            pltpu.sync_copy(acc.at[pl.ds(r, K)], rows_ring.at[0])
            pltpu.sync_copy(deg.at[pl.ds(r, K)], degc_v)

            def normg(g, c2):
                nv = 1.0 / jnp.maximum(degc_v[pl.ds(g * 16, 16)], 1.0)
                for l in range(16):
                    s = nv[l]
                    k = g * 16 + l
                    for j in range(D // 16):
                        sl = pl.ds(j * 16, 16)
                        rows_ring[0, k, sl] = rows_ring[0, k, sl] * s
                return c2

            lax.fori_loop(0, K // 16, normg, 0)
            pltpu.sync_copy(rows_ring.at[0], sio.at[pl.ds(cid * P + r, K)])
            return carry

        lax.fori_loop(0, ROWS_T // K, flushc, 0)
        pltpu.sync_copy(z2d.at[pl.ds(rows0, ROWS_T)],
                        acc.at[pl.ds(rows0, ROWS_T)])
        plsc.subcore_barrier()

        pbase2 = wid * NBLK2

        def pack2_issue(q, b):
            pltpu.async_copy(pack2.at[pbase2 + b], idx_ring.at[q], psems[q])
            pltpu.async_copy(wpack.at[pbase2 + b], w_ring.at[q], psems[q])

        def pack2_drain(q):
            pltpu.make_async_copy(pack2.at[pbase2], idx_ring.at[q],
                                  psems[q]).wait()
            pltpu.make_async_copy(wpack.at[pbase2], w_ring.at[q],
                                  psems[q]).wait()

        def mulw(q, r):
            def body(g, c2):
                wv = w_ring[q, pl.ds(g * 16, 16)]
                for l in range(16):
                    s = wv[l]
                    k = g * 16 + l
                    for j in range(D // 16):
                        sl = pl.ds(j * 16, 16)
                        rows_ring[r, k, sl] = rows_ring[r, k, sl] * s
                return c2

            lax.fori_loop(0, K // 16, body, 0)

        def it2(b, u, first, last_issue, last_pack):
            r0, r1 = u % NROW, (u + 1) % NROW
            q0, q1, q2 = u % NIDX, (u + 1) % NIDX, (u + 2) % NIDX
            if not first:
                drain_rows(ssems, r1)
            if not last_issue:
                pack2_drain(q1)
                gather1(q1, r1)
            drain_rows(gsems, r0)
            mulw(q0, r0)
            pltpu.async_copy(rows_ring.at[r0], acc.at[idx_ring.at[q0, 1]],
                             ssems[r0], add=True)
            if not last_pack:
                pack2_issue(q2, b + 2)

        pack2_issue(0, 0)
        pack2_issue(1, 1)
        pack2_drain(0)
        gather1(0, 0)
        it2(0, 0, True, False, False)
        it2(1, 1, True, False, False)

        M2 = (NBLK2 - 4) // 12 * 12

        def body2(b2, carry):
            base = 2 + b2 * 12
            for u in range(12):
                it2(base + u, 2 + u, False, False, False)
            return carry

        lax.fori_loop(0, M2 // 12, body2, 0)
        for b in range(2 + M2, NBLK2):
            it2(b, b, False, b + 1 >= NBLK2, b + 2 >= NBLK2)
        for b in (NBLK2 - 2, NBLK2 - 1):
            drain_rows(ssems, b % NROW)
        plsc.subcore_barrier()

        pltpu.sync_copy(acc.at[pl.ds(rows0, ROWS_T)],
                        jpart.at[pl.ds(cid * P + rows0, ROWS_T)])

    return sc_pass


def _tc_dense(emb, sio, jpart, W_in, W_out, W_loop, W_jump, loop_rel, jw):
    R = 1024
    num_e = emb.shape[0]
    grid = (P // R,)
    hi = jax.lax.Precision.HIGHEST

    def body(jw_ref, emb_ref, sin_ref, sout_ref, j0_ref, j1_ref, wi_ref,
             wo_ref, wl_ref, wjm_ref, lr_ref, out_ref):
        acc = jnp.dot(sin_ref[...], wi_ref[...], precision=hi)
        acc = acc + jnp.dot(sout_ref[...], wo_ref[...], precision=hi)
        acc = acc + jnp.dot(emb_ref[...] - lr_ref[...], wl_ref[...],
                            precision=hi)
        emb2 = jnp.tanh(acc * (1.0 / 3.0))
        jr = jnp.dot(j0_ref[...] + j1_ref[...], wjm_ref[...], precision=hi)
        out_ref[...] = emb2 + jw_ref[0] * jr

    blk = lambda im: pl.BlockSpec((R, D), im)
    wspec = pl.BlockSpec((D, D), lambda i: (0, 0))
    return pl.pallas_call(
        body,
        grid=grid,
        in_specs=[
            pl.BlockSpec(memory_space=pltpu.SMEM),
            blk(lambda i: (i, 0)),
            blk(lambda i: (i, 0)),
            blk(lambda i: (i + grid[0], 0)),
            blk(lambda i: (i, 0)),
            blk(lambda i: (i + grid[0], 0)),
            wspec, wspec, wspec, wspec,
            pl.BlockSpec((1, D), lambda i: (0, 0)),
        ],
        out_specs=blk(lambda i: (i, 0)),
        out_shape=jax.ShapeDtypeStruct((num_e, D), jnp.float32),
    )(jw, emb, sio, sio, jpart, jpart, W_in, W_out, W_loop, W_jump,
      loop_rel)


def kernel(t, emb, change, rel_emb, W_in, W_out, W_loop, loop_rel, W_jump,
           jump_weight, edge_w_jump, edge_index, edge_type, edge_id_jump):
    num_e = emb.shape[0]
    n_jump = edge_id_jump.shape[1]

    src_all = edge_index[0]
    dst_all = edge_index[1]
    nrel = rel_emb.shape[0]
    et_spread = edge_type + nrel * (
        jnp.arange(edge_type.shape[0], dtype=jnp.int32) % NREP)
    pack1 = jnp.stack([src_all.reshape(-1, K), et_spread.reshape(-1, K),
                       dst_all.reshape(-1, K)], axis=1)
    npad = EJP - n_jump
    srcj = jnp.pad(edge_id_jump[0], (0, npad))
    dstj = jnp.concatenate(
        [edge_id_jump[1],
         num_e + (jnp.arange(npad, dtype=jnp.int32) % (P - num_e))])
    wj = jnp.pad(edge_w_jump[:, 0], (0, npad))
    pack2 = jnp.stack([srcj.reshape(-1, K), dstj.reshape(-1, K),
                       jnp.zeros_like(srcj).reshape(-1, K)], axis=1)
    wpack = wj.reshape(-1, K)
    negrel = jnp.tile(-rel_emb, (NREP, 1))
    ones_h = jnp.ones((K,), jnp.float32)
    z2d = jnp.zeros((P, D), jnp.float32)
    z1d = jnp.zeros((P,), jnp.float32)

    sc = _sc_segment_sums()
    sio, jpart = sc(pack1, pack2, wpack, emb, negrel, ones_h, z2d, z1d)

    dchange = _tc_dense(emb, sio, jpart, W_in, W_out, W_loop, W_jump,
                        loop_rel, jump_weight)
    return (change, dchange)


# speedup vs baseline: 1.1045x; 1.1045x over previous
import functools

import jax
import jax.numpy as jnp
from jax import lax
from jax.experimental import pallas as pl
from jax.experimental.pallas import tpu as pltpu
from jax.experimental.pallas import tpu_sc as plsc

NC = 2
NS = 16
D = 128
P = 10240
ROWS_T = P // NS
K = 80
NROW = 2
NIDX = 4
E1T = 10000
E2T = 5120
EJP = NC * NS * E2T
NBLK1 = E1T // K
NBLK2 = E2T // K
NREP = 32


def _sc_segment_sums():
    mesh = plsc.VectorSubcoreMesh(
        core_axis_name="c", subcore_axis_name="s", num_cores=NC,
        num_subcores=NS)

    @functools.partial(
        pl.kernel,
        mesh=mesh,
        out_type=(
            jax.ShapeDtypeStruct((NC * P, D), jnp.float32),
            jax.ShapeDtypeStruct((NC * P, D), jnp.float32),
        ),
        scratch_types=[
            pltpu.VMEM_SHARED((P, D), jnp.float32),
            pltpu.VMEM_SHARED((P,), jnp.float32),
            pltpu.VMEM((NIDX, 3, K), jnp.int32),
            pltpu.VMEM((NIDX, K), jnp.float32),
            pltpu.VMEM((NROW, K, D), jnp.float32),
            pltpu.VMEM((NROW, K, D), jnp.float32),
            pltpu.VMEM((K,), jnp.float32),
            pltpu.VMEM((K,), jnp.float32),
            [pltpu.SemaphoreType.DMA] * NROW,
            [pltpu.SemaphoreType.DMA] * NROW,
            [pltpu.SemaphoreType.DMA] * NIDX,
        ],
    )
    def sc_pass(pack1, pack2, wpack, emb_h, negrel_h,
                ones_h, z2d, z1d, sio, jpart, acc, deg,
                idx_ring, w_ring, rows_ring, rel_ring, ones_v,
                degc_v, gsems, ssems, psems):
        cid = lax.axis_index("c")
        sid = lax.axis_index("s")
        wid = cid * NS + sid
        rows0 = sid * ROWS_T

        pltpu.sync_copy(ones_h, ones_v)
        pltpu.sync_copy(z2d.at[pl.ds(rows0, ROWS_T)],
                        acc.at[pl.ds(rows0, ROWS_T)])
        pltpu.sync_copy(z1d.at[pl.ds(rows0, ROWS_T)],
                        deg.at[pl.ds(rows0, ROWS_T)])
        plsc.subcore_barrier()

        def drain_rows(sems, r):
            pltpu.make_async_copy(z2d.at[pl.ds(0, K)], rows_ring.at[r],
                                  sems[r]).wait()

        def drain_ones(r):
            pltpu.make_async_copy(z1d.at[pl.ds(0, K)], degc_v,
                                  ssems[r]).wait()

        def drain_rel(sems, r):
            pltpu.make_async_copy(z2d.at[pl.ds(0, K)], rel_ring.at[r],
                                  sems[r]).wait()

        pbase1 = wid * NBLK1

        def pack1_issue(q, b):
            pltpu.async_copy(pack1.at[pbase1 + b], idx_ring.at[q], psems[q])

        def pack1_drain(q):
            pltpu.make_async_copy(pack1.at[pbase1], idx_ring.at[q],
                                  psems[q]).wait()

        def gather1(q, r):
            pltpu.async_copy(emb_h.at[idx_ring.at[q, 0]], rows_ring.at[r],
                             gsems[r])
            pltpu.async_copy(negrel_h.at[idx_ring.at[q, 1]], rel_ring.at[r],
                             gsems[r])

        def scatter1(q, r):
            pltpu.async_copy(rows_ring.at[r], acc.at[idx_ring.at[q, 2]],
                             ssems[r], add=True)
            pltpu.async_copy(rel_ring.at[r], acc.at[idx_ring.at[q, 2]],
                             ssems[r], add=True)
            pltpu.async_copy(ones_v, deg.at[idx_ring.at[q, 2]],
                             ssems[r], add=True)

        def it1(b, u, first, last_issue, last_pack):
            r0, r1 = u % NROW, (u + 1) % NROW
            q0, q1, q2 = u % NIDX, (u + 1) % NIDX, (u + 2) % NIDX
            if not first:
                drain_rows(ssems, r1)
                drain_rel(ssems, r1)
                drain_ones(r1)
            if not last_issue:
                pack1_drain(q1)
                gather1(q1, r1)
            drain_rows(gsems, r0)
            drain_rel(gsems, r0)
            scatter1(q0, r0)
            if not last_pack:
                pack1_issue(q2, b + 2)

        pack1_issue(0, 0)
        pack1_issue(1, 1)
        pack1_drain(0)
        gather1(0, 0)
        it1(0, 0, True, False, False)
        it1(1, 1, False, False, False)

        M1 = (NBLK1 - 4) // 4 * 4

        def body1(b2, carry):
            base = 2 + b2 * 4
            for u in range(4):
                it1(base + u, 2 + u, False, False, False)
            return carry

        lax.fori_loop(0, M1 // 4, body1, 0)
        for b in range(2 + M1, NBLK1):
            it1(b, b, False, b + 1 >= NBLK1, b + 2 >= NBLK1)
        drain_rows(ssems, (NBLK1 - 1) % NROW)
        drain_rel(ssems, (NBLK1 - 1) % NROW)
        drain_ones((NBLK1 - 1) % NROW)
        plsc.subcore_barrier()

        def flushc(c, carry):
            r = rows0 + c * K
            pltpu.sync_copy(acc.at[pl.ds(r, K)], rows_ring.at[0])
            pltpu.sync_copy(deg.at[pl.ds(r, K)], degc_v)

            def normg(g, c2):
                nv = 1.0 / jnp.maximum(degc_v[pl.ds(g * 16, 16)], 1.0)
                for l in range(16):
                    s = nv[l]
                    k = g * 16 + l
                    for j in range(D // 16):
                        sl = pl.ds(j * 16, 16)
                        rows_ring[0, k, sl] = rows_ring[0, k, sl] * s
                return c2

            lax.fori_loop(0, K // 16, normg, 0)
            pltpu.sync_copy(rows_ring.at[0], sio.at[pl.ds(cid * P + r, K)])
            return carry

        lax.fori_loop(0, ROWS_T // K, flushc, 0)
        pltpu.sync_copy(z2d.at[pl.ds(rows0, ROWS_T)],
                        acc.at[pl.ds(rows0, ROWS_T)])
        plsc.subcore_barrier()

        pbase2 = wid * NBLK2

        def pack2_issue(q, b):
            pltpu.async_copy(pack2.at[pbase2 + b], idx_ring.at[q], psems[q])
            pltpu.async_copy(wpack.at[pbase2 + b], w_ring.at[q], psems[q])

        def pack2_drain(q):
            pltpu.make_async_copy(pack2.at[pbase2], idx_ring.at[q],
                                  psems[q]).wait()
            pltpu.make_async_copy(wpack.at[pbase2], w_ring.at[q],
                                  psems[q]).wait()

        def mulw(q, r):
            def body(g, c2):
                wv = w_ring[q, pl.ds(g * 16, 16)]
                for l in range(16):
                    s = wv[l]
                    k = g * 16 + l
                    for j in range(D // 16):
                        sl = pl.ds(j * 16, 16)
                        rows_ring[r, k, sl] = rows_ring[r, k, sl] * s
                return c2

            lax.fori_loop(0, K // 16, body, 0)

        def gather2(q, r):
            pltpu.async_copy(emb_h.at[idx_ring.at[q, 0]], rows_ring.at[r],
                             gsems[r])

        def it2(b, u, first, last_issue, last_pack):
            r0, r1 = u % NROW, (u + 1) % NROW
            q0, q1, q2 = u % NIDX, (u + 1) % NIDX, (u + 2) % NIDX
            if not first:
                drain_rows(ssems, r1)
            if not last_issue:
                pack2_drain(q1)
                gather2(q1, r1)
            drain_rows(gsems, r0)
            mulw(q0, r0)
            pltpu.async_copy(rows_ring.at[r0], acc.at[idx_ring.at[q0, 1]],
                             ssems[r0], add=True)
            if not last_pack:
                pack2_issue(q2, b + 2)

        pack2_issue(0, 0)
        pack2_issue(1, 1)
        pack2_drain(0)
        gather2(0, 0)
        it2(0, 0, True, False, False)
        it2(1, 1, False, False, False)

        M2 = (NBLK2 - 4) // 4 * 4

        def body2(b2, carry):
            base = 2 + b2 * 4
            for u in range(4):
                it2(base + u, 2 + u, False, False, False)
            return carry

        lax.fori_loop(0, M2 // 4, body2, 0)
        for b in range(2 + M2, NBLK2):
            it2(b, b, False, b + 1 >= NBLK2, b + 2 >= NBLK2)
        drain_rows(ssems, (NBLK2 - 1) % NROW)
        plsc.subcore_barrier()

        pltpu.sync_copy(acc.at[pl.ds(rows0, ROWS_T)],
                        jpart.at[pl.ds(cid * P + rows0, ROWS_T)])

    return sc_pass


def _tc_dense(emb, sio, jpart, W_in, W_out, W_loop, W_jump, loop_rel, jw):
    R = 1024
    num_e = emb.shape[0]
    grid = (P // R,)
    hi = jax.lax.Precision.HIGHEST

    def body(jw_ref, emb_ref, sin_ref, sout_ref, j0_ref, j1_ref, wi_ref,
             wo_ref, wl_ref, wjm_ref, lr_ref, out_ref):
        acc = jnp.dot(sin_ref[...], wi_ref[...], precision=hi)
        acc = acc + jnp.dot(sout_ref[...], wo_ref[...], precision=hi)
        acc = acc + jnp.dot(emb_ref[...] - lr_ref[...], wl_ref[...],
                            precision=hi)
        emb2 = jnp.tanh(acc * (1.0 / 3.0))
        jr = jnp.dot(j0_ref[...] + j1_ref[...], wjm_ref[...], precision=hi)
        out_ref[...] = emb2 + jw_ref[0] * jr

    blk = lambda im: pl.BlockSpec((R, D), im)
    wspec = pl.BlockSpec((D, D), lambda i: (0, 0))
    return pl.pallas_call(
        body,
        grid=grid,
        in_specs=[
            pl.BlockSpec(memory_space=pltpu.SMEM),
            blk(lambda i: (i, 0)),
            blk(lambda i: (i, 0)),
            blk(lambda i: (i + grid[0], 0)),
            blk(lambda i: (i, 0)),
            blk(lambda i: (i + grid[0], 0)),
            wspec, wspec, wspec, wspec,
            pl.BlockSpec((1, D), lambda i: (0, 0)),
        ],
        out_specs=blk(lambda i: (i, 0)),
        out_shape=jax.ShapeDtypeStruct((num_e, D), jnp.float32),
    )(jw, emb, sio, sio, jpart, jpart, W_in, W_out, W_loop, W_jump,
      loop_rel)


def kernel(t, emb, change, rel_emb, W_in, W_out, W_loop, loop_rel, W_jump,
           jump_weight, edge_w_jump, edge_index, edge_type, edge_id_jump):
    num_e = emb.shape[0]
    n_jump = edge_id_jump.shape[1]

    src_all = edge_index[0]
    dst_all = edge_index[1]
    nrel = rel_emb.shape[0]
    et_spread = edge_type + nrel * (
        jnp.arange(edge_type.shape[0], dtype=jnp.int32) % NREP)
    pack1 = jnp.stack([src_all.reshape(-1, K), et_spread.reshape(-1, K),
                       dst_all.reshape(-1, K)], axis=1)
    npad = EJP - n_jump
    srcj = jnp.pad(edge_id_jump[0], (0, npad))
    dstj = jnp.concatenate(
        [edge_id_jump[1],
         num_e + (jnp.arange(npad, dtype=jnp.int32) % (P - num_e))])
    wj = jnp.pad(edge_w_jump[:, 0], (0, npad))
    pack2 = jnp.stack([srcj.reshape(-1, K), dstj.reshape(-1, K),
                       jnp.zeros_like(srcj).reshape(-1, K)], axis=1)
    wpack = wj.reshape(-1, K)
    negrel = jnp.tile(-rel_emb, (NREP, 1))
    ones_h = jnp.ones((K,), jnp.float32)
    z2d = jnp.zeros((P, D), jnp.float32)
    z1d = jnp.zeros((P,), jnp.float32)

    sc = _sc_segment_sums()
    sio, jpart = sc(pack1, pack2, wpack, emb, negrel, ones_h, z2d, z1d)

    dchange = _tc_dense(emb, sio, jpart, W_in, W_out, W_loop, W_jump,
                        loop_rel, jump_weight)
    return (change, dchange)
